# baseline (speedup 1.0000x reference)
import functools

import jax
import jax.numpy as jnp
import numpy as np
from jax import lax
from jax.experimental import pallas as pl
from jax.experimental.pallas import tpu as pltpu
from jax.experimental.pallas import tpu_sc as plsc

N = 10000
E = 160000
D_IN = 92
D = 256
H = 128
BINS = 256
G = 64
NC = 2
NS = 16
L = 16
NP = 10240
CA = 80
CC = 80
CB = 800
CD = 128
INV_SQRT_D = 1.0 / 16.0

_SC_PARAMS = pltpu.CompilerParams(needs_layout_passes=False)


def _sc_mesh():
    return plsc.VectorSubcoreMesh(core_axis_name="c", subcore_axis_name="s",
                                  num_cores=NC, num_subcores=NS)


def _embed_body(x_ref, aw1_ref, ab1_ref, aw2_ref, ab2_ref, pw_ref, pb_ref,
                af_ref, ppe_ref):
    x = x_ref[...]
    fx = jnp.floor(x)
    af = jnp.dot(fx, aw1_ref[...], preferred_element_type=jnp.float32) + ab1_ref[...]
    af = af * jax.nn.sigmoid(af)
    af_ref[...] = jnp.dot(af, aw2_ref[...], preferred_element_type=jnp.float32) + ab2_ref[...]
    ppe = (x - jnp.trunc(x)) * 1000.0
    ppe_ref[...] = jnp.dot(ppe, pw_ref[...], preferred_element_type=jnp.float32) + pb_ref[...]


def _embed(x, params, block_m=1000):
    grid = (N // block_m,)
    row = lambda i: (i, 0)
    rep2 = lambda i: (0, 0)
    rep1 = lambda i: (0,)
    return pl.pallas_call(
        _embed_body,
        grid=grid,
        in_specs=[
            pl.BlockSpec((block_m, D_IN), row),
            pl.BlockSpec((D_IN, D), rep2),
            pl.BlockSpec((D,), rep1),
            pl.BlockSpec((D, D), rep2),
            pl.BlockSpec((D,), rep1),
            pl.BlockSpec((D_IN, D), rep2),
            pl.BlockSpec((D,), rep1),
        ],
        out_specs=[pl.BlockSpec((block_m, D), row), pl.BlockSpec((block_m, D), row)],
        out_shape=[jax.ShapeDtypeStruct((N, D), jnp.float32),
                   jax.ShapeDtypeStruct((N, D), jnp.float32)],
    )(x, params['atom_W1'], params['atom_b1'], params['atom_W2'],
      params['atom_b2'], params['ppe_W'], params['ppe_b'])


def _edge_feat_body(ea_ref, w_ref, b_ref, o_ref, *, gamma):
    ea = ea_ref[...]
    d = -1.0 / jnp.sqrt(jnp.sum(ea * ea, axis=1, keepdims=True))
    centers = -6.0 + (6.0 / (BINS - 1)) * lax.broadcasted_iota(
        jnp.int32, (1, BINS), 1).astype(jnp.float32)
    rbf = jnp.exp(-gamma * (d - centers) ** 2)
    ef = jnp.dot(rbf, w_ref[...], preferred_element_type=jnp.float32) + b_ref[...]
    o_ref[...] = ef * jax.nn.sigmoid(ef)


def _edge_feat(edge_attr, params, block_m=1000):
    gamma = float(1.0 / np.diff(np.linspace(-6.0, 0.0, BINS)).mean())
    grid = (E // block_m,)
    return pl.pallas_call(
        functools.partial(_edge_feat_body, gamma=gamma),
        grid=grid,
        in_specs=[
            pl.BlockSpec((block_m, 3), lambda i: (i, 0)),
            pl.BlockSpec((BINS, D), lambda i: (0, 0)),
            pl.BlockSpec((D,), lambda i: (0,)),
        ],
        out_specs=pl.BlockSpec((block_m, D), lambda i: (i, 0)),
        out_shape=jax.ShapeDtypeStruct((E, D), jnp.float32),
    )(edge_attr, params['edge_W'], params['edge_b'])


def _qkvs_body(h_ref, wq_ref, bq_ref, wk_ref, bk_ref, wv_ref, bv_ref,
               ws_ref, bs_ref, q_ref, k_ref, v_ref, s_ref):
    h = h_ref[...]
    q_ref[...] = jnp.dot(h, wq_ref[...], preferred_element_type=jnp.float32) + bq_ref[...]
    k_ref[...] = jnp.dot(h, wk_ref[...], preferred_element_type=jnp.float32) + bk_ref[...]
    v_ref[...] = jnp.dot(h, wv_ref[...], preferred_element_type=jnp.float32) + bv_ref[...]
    s_ref[...] = jnp.dot(h, ws_ref[...], preferred_element_type=jnp.float32) + bs_ref[...]


def _qkvs(h, p, block_m=1000):
    nb = N // block_m
    grid = (NC, nb)
    hmap = lambda c, i: (i, 0)
    wmap = lambda c, i: (0, c)
    bmap = lambda c, i: (c,)
    omap = lambda c, i: (c * nb + i, 0)
    ospec = pl.BlockSpec((block_m, H), omap)
    oshape = jax.ShapeDtypeStruct((NC * N, H), jnp.float32)
    return pl.pallas_call(
        _qkvs_body,
        grid=grid,
        in_specs=[
            pl.BlockSpec((block_m, D), hmap),
            pl.BlockSpec((D, H), wmap), pl.BlockSpec((H,), bmap),
            pl.BlockSpec((D, H), wmap), pl.BlockSpec((H,), bmap),
            pl.BlockSpec((D, H), wmap), pl.BlockSpec((H,), bmap),
            pl.BlockSpec((D, H), wmap), pl.BlockSpec((H,), bmap),
        ],
        out_specs=[ospec, ospec, ospec, ospec],
        out_shape=[oshape, oshape, oshape, oshape],
    )(h, p['Wq'], p['bq'], p['Wk'], p['bk'], p['Wv'], p['bv'], p['Ws'], p['bs'])


def _esplit_body(ef_ref, w_ref, b_ref, o_ref):
    o_ref[...] = (jnp.dot(ef_ref[...], w_ref[...],
                          preferred_element_type=jnp.float32) + b_ref[...])


def _esplit(ef, We, be, block_m=1000):
    nb = E // block_m
    grid = (NC, nb)
    return pl.pallas_call(
        _esplit_body,
        grid=grid,
        in_specs=[
            pl.BlockSpec((block_m, D), lambda c, i: (i, 0)),
            pl.BlockSpec((D, H), lambda c, i: (0, c)),
            pl.BlockSpec((H,), lambda c, i: (c,)),
        ],
        out_specs=pl.BlockSpec((block_m, H), lambda c, i: (c * nb + i, 0)),
        out_shape=jax.ShapeDtypeStruct((NC * E, H), jnp.float32),
    )(ef, We, be)


def _merge_body(o0_ref, o1_ref, s0_ref, s1_ref, h_ref):
    h_ref[:, :H] = o0_ref[...] + s0_ref[...]
    h_ref[:, H:] = o1_ref[...] + s1_ref[...]


def _merge(o0, o1, s4, block_m=1000):
    nb = N // block_m
    grid = (nb,)
    lo = lambda i: (i, 0)
    hi = lambda i: (nb + i, 0)
    hspec = pl.BlockSpec((block_m, H), lo)
    hspec1 = pl.BlockSpec((block_m, H), hi)
    return pl.pallas_call(
        _merge_body,
        grid=grid,
        in_specs=[hspec, hspec, hspec, hspec1],
        out_specs=pl.BlockSpec((block_m, D), lo),
        out_shape=jax.ShapeDtypeStruct((N, D), jnp.float32),
    )(o0, o1, s4, s4)


def _ppe_body(h_ref, adj_ref, rm_ref, rv_ref, g_ref, bt_ref,
              w1_ref, b1_ref, w2_ref, b2_ref, w3_ref, b3_ref,
              h_out_ref, adj_out_ref):
    h = h_ref[...]
    adj = adj_ref[...] + h
    adj_out_ref[...] = adj
    adjn = (adj - rm_ref[...]) / jnp.sqrt(rv_ref[...] + 1e-5) * g_ref[...] + bt_ref[...]
    adj2 = jnp.dot(adjn, w1_ref[...], preferred_element_type=jnp.float32) + b1_ref[...]
    x1 = adj2[:, :D]
    x2 = adj2[:, D:]
    x1 = jnp.dot(x1, w2_ref[...], preferred_element_type=jnp.float32) + b2_ref[...]
    x2 = 0.5 * x2 * (1.0 + lax.erf(x2 / jnp.sqrt(2.0)))
    h_out_ref[...] = (jnp.dot(x1 * x2, w3_ref[...], preferred_element_type=jnp.float32)
                      + b3_ref[...] + h)


def _ppeconv(p, h, adj, block_m=1000):
    grid = (N // block_m,)
    row = lambda i: (i, 0)
    rep2 = lambda i: (0, 0)
    rep1 = lambda i: (0,)
    return pl.pallas_call(
        _ppe_body,
        grid=grid,
        in_specs=[
            pl.BlockSpec((block_m, D), row),
            pl.BlockSpec((block_m, D), row),
            pl.BlockSpec((D,), rep1),
            pl.BlockSpec((D,), rep1),
            pl.BlockSpec((D,), rep1),
            pl.BlockSpec((D,), rep1),
            pl.BlockSpec((D, 2 * D), rep2),
            pl.BlockSpec((2 * D,), rep1),
            pl.BlockSpec((D, D), rep2),
            pl.BlockSpec((D,), rep1),
            pl.BlockSpec((D, D), rep2),
            pl.BlockSpec((D,), rep1),
        ],
        out_specs=[pl.BlockSpec((block_m, D), row), pl.BlockSpec((block_m, D), row)],
        out_shape=[jax.ShapeDtypeStruct((N, D), jnp.float32),
                   jax.ShapeDtypeStruct((N, D), jnp.float32)],
    )(h, adj, p['rm'], p['rv'], p['g'], p['bt'],
      p['W1'], p['b1'], p['W2'], p['b2'], p['W3'], p['b3'])


def _pool_body(h_ref, seg_ref, fcw_ref, fcb_ref, ow_ref, ob_ref, osda_ref, out_ref):
    h = h_ref[...]
    seg = seg_ref[...]
    one_hot = (seg[:, None] == lax.broadcasted_iota(jnp.int32, (1, G), 1)).astype(jnp.float32)
    sums = jax.lax.dot_general(one_hot, h, (((0,), (0,)), ((), ())),
                               preferred_element_type=jnp.float32)
    cnt = jnp.sum(one_hot, axis=0)
    ft = sums / jnp.maximum(cnt, 1.0)[:, None]
    fc = jnp.dot(ft, fcw_ref[...], preferred_element_type=jnp.float32) + fcb_ref[...]
    cf = ft + fc * jax.nn.sigmoid(fc)
    out = jnp.dot(cf, ow_ref[...], preferred_element_type=jnp.float32) + ob_ref[...]
    out_ref[...] = out[:, 0] / osda_ref[...]


def _pool(h, batch, num_osda, params):
    return pl.pallas_call(
        _pool_body,
        in_specs=[
            pl.BlockSpec((N, D), lambda: (0, 0)),
            pl.BlockSpec((N,), lambda: (0,)),
            pl.BlockSpec((D, D), lambda: (0, 0)),
            pl.BlockSpec((D,), lambda: (0,)),
            pl.BlockSpec((D, 1), lambda: (0, 0)),
            pl.BlockSpec((1,), lambda: (0,)),
            pl.BlockSpec((G,), lambda: (0,)),
        ],
        out_specs=pl.BlockSpec((G,), lambda: (0,)),
        out_shape=jax.ShapeDtypeStruct((G,), jnp.float32),
    )(h, batch, params['fc_W'], params['fc_b'], params['out_W'],
      params['out_b'], num_osda)


def _sc_dots_body(qs, ks, es, src, dst, part,
                  isrc_v, idst_v, ig_v, qbuf, kbuf, ebuf, pbuf, sem_q, sem_k):
    c = lax.axis_index("c")
    s = lax.axis_index("s")
    coff_n = c * N
    coff_e = c * E
    lanes = lax.iota(jnp.int32, L)

    def chunk(i, _):
        t = i * NS + s
        off = pl.multiple_of(t * CA, 8)
        pltpu.sync_copy(dst.at[pl.ds(off, CA)], idst_v)
        pltpu.sync_copy(src.at[pl.ds(off, CA)], isrc_v)

        def addoff(u, _):
            ig_v[pl.ds(u * L, L)] = idst_v[pl.ds(u * L, L)] + coff_n
            return _
        lax.fori_loop(0, CA // L, addoff, None)
        cq = pltpu.async_copy(qs.at[ig_v], qbuf, sem_q)

        def addoff2(u, _):
            ig_v[pl.ds(u * L, L)] = isrc_v[pl.ds(u * L, L)] + coff_n
            return _
        cq.wait()
        lax.fori_loop(0, CA // L, addoff2, None)
        ck = pltpu.async_copy(ks.at[ig_v], kbuf, sem_k)
        pltpu.sync_copy(es.at[pl.ds(pl.multiple_of(coff_e + off, 8), CA)], ebuf)
        ck.wait()

        def grp(g, _):
            rows = g * L + lanes
            acc = jnp.zeros((L,), jnp.float32)
            for j in range(H):
                cj = jnp.full((L,), j, jnp.int32)
                qg = plsc.load_gather(qbuf, [rows, cj])
                kg = plsc.load_gather(kbuf, [rows, cj])
                eg = plsc.load_gather(ebuf, [rows, cj])
                acc = acc + qg * (kg + eg)
            pbuf[pl.ds(g * L, L)] = acc
            return _
        lax.fori_loop(0, CA // L, grp, None)
        pltpu.sync_copy(pbuf, part.at[pl.ds(pl.multiple_of(coff_e + off, 8), CA)])
        return _

    lax.fori_loop(0, E // CA // NS, chunk, None)


def _sc_dots(qs, ks, es, src, dst):
    return functools.partial(
        pl.kernel,
        out_type=jax.ShapeDtypeStruct((NC * E,), jnp.float32),
        mesh=_sc_mesh(),
        scratch_types=[
            pltpu.VMEM((CA,), jnp.int32),
            pltpu.VMEM((CA,), jnp.int32),
            pltpu.VMEM((CA,), jnp.int32),
            pltpu.VMEM((CA, H), jnp.float32),
            pltpu.VMEM((CA, H), jnp.float32),
            pltpu.VMEM((CA, H), jnp.float32),
            pltpu.VMEM((CA,), jnp.float32),
            pltpu.SemaphoreType.DMA,
            pltpu.SemaphoreType.DMA,
        ],
        compiler_params=_SC_PARAMS,
    )(_sc_dots_body)(qs, ks, es, src, dst)


def _sc_logits_max_body(part, dst, logits, mpart, idst_v, p0_v, p1_v, l_v, m_v):
    c = lax.axis_index("c")
    s = lax.axis_index("s")
    wid = s * NC + c

    def initm(u, _):
        m_v[pl.ds(u * L, L)] = jnp.full((L,), -1e30, jnp.float32)
        return _
    lax.fori_loop(0, NP // L, initm, None)

    def chunk(i, _):
        t = i * (NC * NS) + wid
        off = pl.multiple_of(t * CB, 8)
        pltpu.sync_copy(part.at[pl.ds(off, CB)], p0_v)
        pltpu.sync_copy(part.at[pl.ds(pl.multiple_of(E + off, 8), CB)], p1_v)
        pltpu.sync_copy(dst.at[pl.ds(off, CB)], idst_v)

        def vlog(u, _):
            ds_ = pl.ds(u * L, L)
            l_v[ds_] = (p0_v[ds_] + p1_v[ds_]) * INV_SQRT_D
            return _
        lax.fori_loop(0, CB // L, vlog, None)
        pltpu.sync_copy(l_v, logits.at[pl.ds(off, CB)])

        def upd(u, _):
            ds_ = pl.ds(u * L, L)
            iv = idst_v[ds_]
            lv = l_v[ds_]

            def cond(msk):
                return jnp.any(msk)

            def step(msk):
                plsc.store_scatter(m_v, [iv], lv, mask=msk)
                mg = plsc.load_gather(m_v, [iv])
                return jnp.logical_and(msk, mg < lv)

            lax.while_loop(cond, step, plsc.load_gather(m_v, [iv]) < lv)
            return _
        lax.fori_loop(0, CB // L, upd, None)
        return _

    nchunks = (E // CB - wid + NC * NS - 1) // (NC * NS)
    lax.fori_loop(0, nchunks, chunk, None)
    pltpu.sync_copy(m_v, mpart.at[pl.ds(pl.multiple_of(wid * NP, 8), NP)])


def _sc_logits_max(part, dst):
    return functools.partial(
        pl.kernel,
        out_type=[jax.ShapeDtypeStruct((E,), jnp.float32),
                  jax.ShapeDtypeStruct((NC * NS * NP,), jnp.float32)],
        mesh=_sc_mesh(),
        scratch_types=[
            pltpu.VMEM((CB,), jnp.int32),
            pltpu.VMEM((CB,), jnp.float32),
            pltpu.VMEM((CB,), jnp.float32),
            pltpu.VMEM((CB,), jnp.float32),
            pltpu.VMEM((NP,), jnp.float32),
        ],
        compiler_params=_SC_PARAMS,
    )(_sc_logits_max_body)(part, dst)


def _sc_ex_den_body(logits, dst, mpart, ex, denpart,
                    idst_v, l_v, ex_v, m_v, t_v, den_sp, z_v):
    c = lax.axis_index("c")
    s = lax.axis_index("s")
    wid = s * NC + c
    stripe = NP // NS

    def initz(u, _):
        z_v[pl.ds(u * L, L)] = jnp.zeros((L,), jnp.float32)
        return _
    lax.fori_loop(0, stripe // L, initz, None)
    pltpu.sync_copy(z_v, den_sp.at[pl.ds(pl.multiple_of(s * stripe, 8), stripe)])

    pltpu.sync_copy(mpart.at[pl.ds(0, NP)], m_v)
    def mergerow(w, _):
        pltpu.sync_copy(mpart.at[pl.ds(pl.multiple_of(w * NP, 8), NP)], t_v)

        def vmax(u, _):
            ds_ = pl.ds(u * L, L)
            m_v[ds_] = jnp.maximum(m_v[ds_], t_v[ds_])
            return _
        lax.fori_loop(0, NP // L, vmax, None)
        return _
    lax.fori_loop(1, NC * NS, mergerow, None)
    plsc.subcore_barrier()

    def chunk(i, _):
        t = i * (NC * NS) + wid
        off = pl.multiple_of(t * CD, 8)
        pltpu.sync_copy(logits.at[pl.ds(off, CD)], l_v)
        pltpu.sync_copy(dst.at[pl.ds(off, CD)], idst_v)

        def vex(u, _):
            ds_ = pl.ds(u * L, L)
            iv = idst_v[ds_]
            mg = plsc.load_gather(m_v, [iv])
            ex_v[ds_] = jnp.exp(l_v[ds_] - mg)
            return _
        lax.fori_loop(0, CD // L, vex, None)
        pltpu.sync_copy(ex_v, ex.at[pl.ds(off, CD)])
        pltpu.sync_copy(ex_v, den_sp.at[idst_v], add=True)
        return _

    nchunks = (E // CD - wid + NC * NS - 1) // (NC * NS)
    lax.fori_loop(0, nchunks, chunk, None)
    plsc.subcore_barrier()

    @pl.when(s == 0)
    def _():
        pltpu.sync_copy(den_sp, denpart.at[pl.ds(pl.multiple_of(c * NP, 8), NP)])


def _sc_ex_den(logits, dst, mpart):
    return functools.partial(
        pl.kernel,
        out_type=[jax.ShapeDtypeStruct((E,), jnp.float32),
                  jax.ShapeDtypeStruct((NC * NP,), jnp.float32)],
        mesh=_sc_mesh(),
        scratch_types=[
            pltpu.VMEM((CD,), jnp.int32),
            pltpu.VMEM((CD,), jnp.float32),
            pltpu.VMEM((CD,), jnp.float32),
            pltpu.VMEM((NP,), jnp.float32),
            pltpu.VMEM((NP,), jnp.float32),
            pltpu.VMEM_SHARED((NP,), jnp.float32),
            pltpu.VMEM((NP // NS,), jnp.float32),
        ],
        compiler_params=_SC_PARAMS,
    )(_sc_ex_den_body)(logits, dst, mpart)


def _sc_out_body(vs, es, exv, denpart, src, dst, outf,
                 isrc_v, idst_v, ig_v, vbuf, ebuf, ex_v, al_v, den_v, db_v,
                 acc_sp, sem_v):
    c = lax.axis_index("c")
    s = lax.axis_index("s")
    coff_n = c * NP
    coff_e = c * E
    nstripe = NP // NS

    def zrow(r, _):
        for j in range(H // L):
            vbuf[r, pl.ds(j * L, L)] = jnp.zeros((L,), jnp.float32)
        return _
    lax.fori_loop(0, CC, zrow, None)
    for rep in range(nstripe // CC):
        pltpu.sync_copy(vbuf, acc_sp.at[pl.ds(
            pl.multiple_of(s * nstripe + rep * CC, 8), CC)])

    def dmerge(u, _):
        doff = pl.multiple_of(u * nstripe, 8)
        pltpu.sync_copy(denpart.at[pl.ds(doff, nstripe)], db_v)

        def v0(w, _):
            ds_ = pl.ds(w * L, L)
            den_v[pl.ds(pl.multiple_of(u * nstripe, 8) + w * L, L)] = db_v[ds_]
            return _
        lax.fori_loop(0, nstripe // L, v0, None)
        pltpu.sync_copy(denpart.at[pl.ds(pl.multiple_of(NP + doff, 8), nstripe)],
                        db_v)

        def v1(w, _):
            ds_ = pl.ds(pl.multiple_of(u * nstripe, 8) + w * L, L)
            den_v[ds_] = den_v[ds_] + db_v[pl.ds(w * L, L)]
            return _
        lax.fori_loop(0, nstripe // L, v1, None)
        return _
    lax.fori_loop(0, NP // nstripe, dmerge, None)
    plsc.subcore_barrier()

    def chunk(i, _):
        t = i * NS + s
        off = pl.multiple_of(t * CC, 8)
        pltpu.sync_copy(dst.at[pl.ds(off, CC)], idst_v)
        pltpu.sync_copy(src.at[pl.ds(off, CC)], isrc_v)

        def addoff(u, _):
            ig_v[pl.ds(u * L, L)] = isrc_v[pl.ds(u * L, L)] + c * N
            return _
        lax.fori_loop(0, CC // L, addoff, None)
        cv = pltpu.async_copy(vs.at[ig_v], vbuf, sem_v)
        pltpu.sync_copy(es.at[pl.ds(pl.multiple_of(coff_e + off, 8), CC)], ebuf)
        pltpu.sync_copy(exv.at[pl.ds(off, CC)], ex_v)

        def valpha(u, _):
            ds_ = pl.ds(u * L, L)
            iv = idst_v[ds_]
            dg = plsc.load_gather(den_v, [iv])
            al_v[ds_] = ex_v[ds_] / (dg + 1e-16)
            return _
        lax.fori_loop(0, CC // L, valpha, None)
        cv.wait()
        lanes = lax.iota(jnp.int32, L)

        def grp(g, _):
            rows = g * L + lanes
            a = al_v[pl.ds(g * L, L)]
            for j in range(H):
                cj = jnp.full((L,), j, jnp.int32)
                vg = plsc.load_gather(vbuf, [rows, cj])
                eg = plsc.load_gather(ebuf, [rows, cj])
                plsc.store_scatter(vbuf, [rows, cj], (vg + eg) * a)
            return _
        lax.fori_loop(0, CC // L, grp, None)
        pltpu.sync_copy(vbuf, acc_sp.at[idst_v], add=True)
        return _

    lax.fori_loop(0, E // CC // NS, chunk, None)
    plsc.subcore_barrier()
    pltpu.sync_copy(acc_sp.at[pl.ds(pl.multiple_of(s * nstripe, 8), nstripe)],
                    outf.at[pl.ds(pl.multiple_of(coff_n + s * nstripe, 8),
                                  nstripe)])


def _sc_out(vs, es, exv, denpart, src, dst):
    return functools.partial(
        pl.kernel,
        out_type=jax.ShapeDtypeStruct((NC * NP, H), jnp.float32),
        mesh=_sc_mesh(),
        scratch_types=[
            pltpu.VMEM((CC,), jnp.int32),
            pltpu.VMEM((CC,), jnp.int32),
            pltpu.VMEM((CC,), jnp.int32),
            pltpu.VMEM((CC, H), jnp.float32),
            pltpu.VMEM((CC, H), jnp.float32),
            pltpu.VMEM((CC,), jnp.float32),
            pltpu.VMEM((CC,), jnp.float32),
            pltpu.VMEM((NP,), jnp.float32),
            pltpu.VMEM((NP // NS,), jnp.float32),
            pltpu.VMEM_SHARED((NP, H), jnp.float32),
            pltpu.SemaphoreType.DMA,
        ],
        compiler_params=_SC_PARAMS,
    )(_sc_out_body)(vs, es, exv, denpart, src, dst)


def _tconv(p, h, ef, src, dst):
    q, k, v, s4 = _qkvs(h, p)
    es = _esplit(ef, p['We'], p['be'])
    part = _sc_dots(q, k, es, src, dst)
    logits, mpart = _sc_logits_max(part, dst)
    ex, denpart = _sc_ex_den(logits, dst, mpart)
    outf = _sc_out(v, es, ex, denpart, src, dst)
    return _merge(outf[:N], outf[NP:NP + N], s4)


def kernel(x, edge_index, edge_attr, batch, num_osda, params):
    af, ppe = _embed(x, params)
    ef = _edge_feat(edge_attr, params)
    src = edge_index[0]
    dst = edge_index[1]
    h = _tconv(params['tc'][0], af, ef, src, dst)
    h, ppe = _ppeconv(params['ppe'][0], h, ppe)
    h = _tconv(params['tc'][1], h, ef, src, dst)
    h, ppe = _ppeconv(params['ppe'][1], h, ppe)
    h = _tconv(params['tc'][2], h, ef, src, dst)
    h, ppe = _ppeconv(params['ppe'][2], h, ppe)
    h = _tconv(params['tc'][3], h, ef, src, dst)
    return _pool(h, batch, num_osda, params)


# speedup vs baseline: 3.0939x; 3.0939x over previous
import functools

import jax
import jax.numpy as jnp
import numpy as np
from jax import lax
from jax.experimental import pallas as pl
from jax.experimental.pallas import tpu as pltpu
from jax.experimental.pallas import tpu_sc as plsc

N = 10000
E = 160000
D_IN = 92
D = 256
H = 128
BINS = 256
G = 64
NC = 2
NS = 16
L = 16
NP = 10240
CA = 40
CC = 40
CB = 800
CD = 128
INV_SQRT_D = 1.0 / 16.0

_SC_PARAMS = pltpu.CompilerParams(needs_layout_passes=False)


def _sc_mesh():
    return plsc.VectorSubcoreMesh(core_axis_name="c", subcore_axis_name="s",
                                  num_cores=NC, num_subcores=NS)


def _embed_body(x_ref, aw1_ref, ab1_ref, aw2_ref, ab2_ref, pw_ref, pb_ref,
                af_ref, ppe_ref):
    x = x_ref[...]
    fx = jnp.floor(x)
    af = jnp.dot(fx, aw1_ref[...], preferred_element_type=jnp.float32) + ab1_ref[...]
    af = af * jax.nn.sigmoid(af)
    af_ref[...] = jnp.dot(af, aw2_ref[...], preferred_element_type=jnp.float32) + ab2_ref[...]
    ppe = (x - jnp.trunc(x)) * 1000.0
    ppe_ref[...] = jnp.dot(ppe, pw_ref[...], preferred_element_type=jnp.float32) + pb_ref[...]


def _embed(x, params, block_m=1000):
    grid = (N // block_m,)
    row = lambda i: (i, 0)
    rep2 = lambda i: (0, 0)
    rep1 = lambda i: (0,)
    return pl.pallas_call(
        _embed_body,
        grid=grid,
        in_specs=[
            pl.BlockSpec((block_m, D_IN), row),
            pl.BlockSpec((D_IN, D), rep2),
            pl.BlockSpec((D,), rep1),
            pl.BlockSpec((D, D), rep2),
            pl.BlockSpec((D,), rep1),
            pl.BlockSpec((D_IN, D), rep2),
            pl.BlockSpec((D,), rep1),
        ],
        out_specs=[pl.BlockSpec((block_m, D), row), pl.BlockSpec((block_m, D), row)],
        out_shape=[jax.ShapeDtypeStruct((N, D), jnp.float32),
                   jax.ShapeDtypeStruct((N, D), jnp.float32)],
    )(x, params['atom_W1'], params['atom_b1'], params['atom_W2'],
      params['atom_b2'], params['ppe_W'], params['ppe_b'])


def _edge_feat_body(ea_ref, w_ref, b_ref, o_ref, *, gamma):
    ea = ea_ref[...]
    d = -1.0 / jnp.sqrt(jnp.sum(ea * ea, axis=1, keepdims=True))
    centers = -6.0 + (6.0 / (BINS - 1)) * lax.broadcasted_iota(
        jnp.int32, (1, BINS), 1).astype(jnp.float32)
    rbf = jnp.exp(-gamma * (d - centers) ** 2)
    ef = jnp.dot(rbf, w_ref[...], preferred_element_type=jnp.float32) + b_ref[...]
    o_ref[...] = ef * jax.nn.sigmoid(ef)


def _edge_feat(edge_attr, params, block_m=1000):
    gamma = float(1.0 / np.diff(np.linspace(-6.0, 0.0, BINS)).mean())
    grid = (E // block_m,)
    return pl.pallas_call(
        functools.partial(_edge_feat_body, gamma=gamma),
        grid=grid,
        in_specs=[
            pl.BlockSpec((block_m, 3), lambda i: (i, 0)),
            pl.BlockSpec((BINS, D), lambda i: (0, 0)),
            pl.BlockSpec((D,), lambda i: (0,)),
        ],
        out_specs=pl.BlockSpec((block_m, D), lambda i: (i, 0)),
        out_shape=jax.ShapeDtypeStruct((E, D), jnp.float32),
    )(edge_attr, params['edge_W'], params['edge_b'])


def _qkvs_body(h_ref, wq_ref, bq_ref, wk_ref, bk_ref, wv_ref, bv_ref,
               ws_ref, bs_ref, q_ref, k_ref, v_ref, s_ref):
    h = h_ref[...]
    q_ref[...] = jnp.dot(h, wq_ref[...], preferred_element_type=jnp.float32) + bq_ref[...]
    k_ref[...] = jnp.dot(h, wk_ref[...], preferred_element_type=jnp.float32) + bk_ref[...]
    v_ref[...] = jnp.dot(h, wv_ref[...], preferred_element_type=jnp.float32) + bv_ref[...]
    s_ref[...] = jnp.dot(h, ws_ref[...], preferred_element_type=jnp.float32) + bs_ref[...]


def _qkvs(h, p, block_m=1000):
    nb = N // block_m
    grid = (NC, nb)
    hmap = lambda c, i: (i, 0)
    wmap = lambda c, i: (0, c)
    bmap = lambda c, i: (c,)
    omap = lambda c, i: (c * nb + i, 0)
    ospec = pl.BlockSpec((block_m, H), omap)
    oshape = jax.ShapeDtypeStruct((NC * N, H), jnp.float32)
    return pl.pallas_call(
        _qkvs_body,
        grid=grid,
        in_specs=[
            pl.BlockSpec((block_m, D), hmap),
            pl.BlockSpec((D, H), wmap), pl.BlockSpec((H,), bmap),
            pl.BlockSpec((D, H), wmap), pl.BlockSpec((H,), bmap),
            pl.BlockSpec((D, H), wmap), pl.BlockSpec((H,), bmap),
            pl.BlockSpec((D, H), wmap), pl.BlockSpec((H,), bmap),
        ],
        out_specs=[ospec, ospec, ospec, ospec],
        out_shape=[oshape, oshape, oshape, oshape],
    )(h, p['Wq'], p['bq'], p['Wk'], p['bk'], p['Wv'], p['bv'], p['Ws'], p['bs'])


def _esplit_body(ef_ref, w_ref, b_ref, o_ref):
    o_ref[...] = (jnp.dot(ef_ref[...], w_ref[...],
                          preferred_element_type=jnp.float32) + b_ref[...])


def _esplit(ef, We, be, block_m=1000):
    nb = E // block_m
    grid = (NC, nb)
    return pl.pallas_call(
        _esplit_body,
        grid=grid,
        in_specs=[
            pl.BlockSpec((block_m, D), lambda c, i: (i, 0)),
            pl.BlockSpec((D, H), lambda c, i: (0, c)),
            pl.BlockSpec((H,), lambda c, i: (c,)),
        ],
        out_specs=pl.BlockSpec((block_m, H), lambda c, i: (c * nb + i, 0)),
        out_shape=jax.ShapeDtypeStruct((NC * E, H), jnp.float32),
    )(ef, We, be)


def _merge_body(o0_ref, o1_ref, s0_ref, s1_ref, h_ref):
    h_ref[:, :H] = o0_ref[...] + s0_ref[...]
    h_ref[:, H:] = o1_ref[...] + s1_ref[...]


def _merge(o0, o1, s4, block_m=1000):
    nb = N // block_m
    grid = (nb,)
    lo = lambda i: (i, 0)
    hi = lambda i: (nb + i, 0)
    hspec = pl.BlockSpec((block_m, H), lo)
    hspec1 = pl.BlockSpec((block_m, H), hi)
    return pl.pallas_call(
        _merge_body,
        grid=grid,
        in_specs=[hspec, hspec, hspec, hspec1],
        out_specs=pl.BlockSpec((block_m, D), lo),
        out_shape=jax.ShapeDtypeStruct((N, D), jnp.float32),
    )(o0, o1, s4, s4)


def _ppe_body(h_ref, adj_ref, rm_ref, rv_ref, g_ref, bt_ref,
              w1_ref, b1_ref, w2_ref, b2_ref, w3_ref, b3_ref,
              h_out_ref, adj_out_ref):
    h = h_ref[...]
    adj = adj_ref[...] + h
    adj_out_ref[...] = adj
    adjn = (adj - rm_ref[...]) / jnp.sqrt(rv_ref[...] + 1e-5) * g_ref[...] + bt_ref[...]
    adj2 = jnp.dot(adjn, w1_ref[...], preferred_element_type=jnp.float32) + b1_ref[...]
    x1 = adj2[:, :D]
    x2 = adj2[:, D:]
    x1 = jnp.dot(x1, w2_ref[...], preferred_element_type=jnp.float32) + b2_ref[...]
    x2 = 0.5 * x2 * (1.0 + lax.erf(x2 / jnp.sqrt(2.0)))
    h_out_ref[...] = (jnp.dot(x1 * x2, w3_ref[...], preferred_element_type=jnp.float32)
                      + b3_ref[...] + h)


def _ppeconv(p, h, adj, block_m=1000):
    grid = (N // block_m,)
    row = lambda i: (i, 0)
    rep2 = lambda i: (0, 0)
    rep1 = lambda i: (0,)
    return pl.pallas_call(
        _ppe_body,
        grid=grid,
        in_specs=[
            pl.BlockSpec((block_m, D), row),
            pl.BlockSpec((block_m, D), row),
            pl.BlockSpec((D,), rep1),
            pl.BlockSpec((D,), rep1),
            pl.BlockSpec((D,), rep1),
            pl.BlockSpec((D,), rep1),
            pl.BlockSpec((D, 2 * D), rep2),
            pl.BlockSpec((2 * D,), rep1),
            pl.BlockSpec((D, D), rep2),
            pl.BlockSpec((D,), rep1),
            pl.BlockSpec((D, D), rep2),
            pl.BlockSpec((D,), rep1),
        ],
        out_specs=[pl.BlockSpec((block_m, D), row), pl.BlockSpec((block_m, D), row)],
        out_shape=[jax.ShapeDtypeStruct((N, D), jnp.float32),
                   jax.ShapeDtypeStruct((N, D), jnp.float32)],
    )(h, adj, p['rm'], p['rv'], p['g'], p['bt'],
      p['W1'], p['b1'], p['W2'], p['b2'], p['W3'], p['b3'])


def _pool_body(h_ref, seg_ref, fcw_ref, fcb_ref, ow_ref, ob_ref, osda_ref, out_ref):
    h = h_ref[...]
    seg = seg_ref[...]
    one_hot = (seg[:, None] == lax.broadcasted_iota(jnp.int32, (1, G), 1)).astype(jnp.float32)
    sums = jax.lax.dot_general(one_hot, h, (((0,), (0,)), ((), ())),
                               preferred_element_type=jnp.float32)
    cnt = jnp.sum(one_hot, axis=0)
    ft = sums / jnp.maximum(cnt, 1.0)[:, None]
    fc = jnp.dot(ft, fcw_ref[...], preferred_element_type=jnp.float32) + fcb_ref[...]
    cf = ft + fc * jax.nn.sigmoid(fc)
    out = jnp.dot(cf, ow_ref[...], preferred_element_type=jnp.float32) + ob_ref[...]
    out_ref[...] = out[:, 0] / osda_ref[...]


def _pool(h, batch, num_osda, params):
    return pl.pallas_call(
        _pool_body,
        in_specs=[
            pl.BlockSpec((N, D), lambda: (0, 0)),
            pl.BlockSpec((N,), lambda: (0,)),
            pl.BlockSpec((D, D), lambda: (0, 0)),
            pl.BlockSpec((D,), lambda: (0,)),
            pl.BlockSpec((D, 1), lambda: (0, 0)),
            pl.BlockSpec((1,), lambda: (0,)),
            pl.BlockSpec((G,), lambda: (0,)),
        ],
        out_specs=pl.BlockSpec((G,), lambda: (0,)),
        out_shape=jax.ShapeDtypeStruct((G,), jnp.float32),
    )(h, batch, params['fc_W'], params['fc_b'], params['out_W'],
      params['out_b'], num_osda)


def _sc_dots_body(qs, ks, es, src, dst, part,
                  iq0, ik0, iq1, ik1, q0, k0, e0, p0, q1, k1, e1, p1,
                  sq0, sk0, se0, sq1, sk1, se1):
    c = lax.axis_index("c")
    s = lax.axis_index("s")
    coff_n = c * N
    coff_e = c * E
    lanes = lax.iota(jnp.int32, L)
    sets = ((iq0, ik0, q0, k0, e0, p0, sq0, sk0, se0),
            (iq1, ik1, q1, k1, e1, p1, sq1, sk1, se1))

    def issue(t, bs):
        iq, ik, qb, kb, eb, pb, sq, sk, se = bs
        off = pl.multiple_of(t * CA, 8)
        pltpu.sync_copy(dst.at[pl.ds(off, CA)], iq)
        pltpu.sync_copy(src.at[pl.ds(off, CA)], ik)

        def addoff(u, _):
            ds_ = pl.ds(u * L, L)
            iq[ds_] = iq[ds_] + coff_n
            ik[ds_] = ik[ds_] + coff_n
            return _
        lax.fori_loop(0, CA // L, addoff, None)
        return (pltpu.async_copy(qs.at[iq], qb, sq),
                pltpu.async_copy(ks.at[ik], kb, sk),
                pltpu.async_copy(
                    es.at[pl.ds(pl.multiple_of(coff_e + off, 8), CA)], eb, se))

    def compute(t, bs, handles):
        iq, ik, qb, kb, eb, pb, sq, sk, se = bs
        off = pl.multiple_of(t * CA, 8)
        for h in handles:
            h.wait()

        def grp(g, _):
            res = jnp.zeros((L,), jnp.float32)
            for i2 in range(L):
                r = g * L + i2
                acc = jnp.zeros((L,), jnp.float32)
                for j in range(H // L):
                    ds_ = pl.ds(j * L, L)
                    acc = acc + qb[r, ds_] * (kb[r, ds_] + eb[r, ds_])
                res = jnp.where(lanes == i2, jnp.sum(acc), res)
            pb[pl.ds(g * L, L)] = res
            return _
        lax.fori_loop(0, CA // L, grp, None)
        pltpu.sync_copy(pb, part.at[pl.ds(pl.multiple_of(coff_e + off, 8), CA)])

    nch = E // CA // NS

    def body(i, _):
        t0 = (2 * i) * NS + s
        t1 = (2 * i + 1) * NS + s
        h0 = issue(t0, sets[0])
        h1 = issue(t1, sets[1])
        compute(t0, sets[0], h0)
        compute(t1, sets[1], h1)
        return _

    lax.fori_loop(0, nch // 2, body, None)


def _sc_dots(qs, ks, es, src, dst):
    return functools.partial(
        pl.kernel,
        out_type=jax.ShapeDtypeStruct((NC * E,), jnp.float32),
        mesh=_sc_mesh(),
        scratch_types=[
            pltpu.VMEM((CA,), jnp.int32),
            pltpu.VMEM((CA,), jnp.int32),
            pltpu.VMEM((CA,), jnp.int32),
            pltpu.VMEM((CA,), jnp.int32),
            pltpu.VMEM((CA, H), jnp.float32),
            pltpu.VMEM((CA, H), jnp.float32),
            pltpu.VMEM((CA, H), jnp.float32),
            pltpu.VMEM((CA,), jnp.float32),
            pltpu.VMEM((CA, H), jnp.float32),
            pltpu.VMEM((CA, H), jnp.float32),
            pltpu.VMEM((CA, H), jnp.float32),
            pltpu.VMEM((CA,), jnp.float32),
            pltpu.SemaphoreType.DMA,
            pltpu.SemaphoreType.DMA,
            pltpu.SemaphoreType.DMA,
            pltpu.SemaphoreType.DMA,
            pltpu.SemaphoreType.DMA,
            pltpu.SemaphoreType.DMA,
        ],
        compiler_params=_SC_PARAMS,
    )(_sc_dots_body)(qs, ks, es, src, dst)


def _sc_logits_max_body(part, dst, logits, mpart, idst_v, p0_v, p1_v, l_v, m_v):
    c = lax.axis_index("c")
    s = lax.axis_index("s")
    wid = s * NC + c

    def initm(u, _):
        m_v[pl.ds(u * L, L)] = jnp.full((L,), -1e30, jnp.float32)
        return _
    lax.fori_loop(0, NP // L, initm, None)

    def chunk(i, _):
        t = i * (NC * NS) + wid
        off = pl.multiple_of(t * CB, 8)
        pltpu.sync_copy(part.at[pl.ds(off, CB)], p0_v)
        pltpu.sync_copy(part.at[pl.ds(pl.multiple_of(E + off, 8), CB)], p1_v)
        pltpu.sync_copy(dst.at[pl.ds(off, CB)], idst_v)

        def vlog(u, _):
            ds_ = pl.ds(u * L, L)
            l_v[ds_] = (p0_v[ds_] + p1_v[ds_]) * INV_SQRT_D
            return _
        lax.fori_loop(0, CB // L, vlog, None)
        pltpu.sync_copy(l_v, logits.at[pl.ds(off, CB)])

        def upd(u, _):
            ds_ = pl.ds(u * L, L)
            iv = idst_v[ds_]
            lv = l_v[ds_]

            def cond(msk):
                return jnp.any(msk)

            def step(msk):
                plsc.store_scatter(m_v, [iv], lv, mask=msk)
                mg = plsc.load_gather(m_v, [iv])
                return jnp.logical_and(msk, mg < lv)

            lax.while_loop(cond, step, plsc.load_gather(m_v, [iv]) < lv)
            return _
        lax.fori_loop(0, CB // L, upd, None)
        return _

    nchunks = (E // CB - wid + NC * NS - 1) // (NC * NS)
    lax.fori_loop(0, nchunks, chunk, None)
    pltpu.sync_copy(m_v, mpart.at[pl.ds(pl.multiple_of(wid * NP, 8), NP)])


def _sc_logits_max(part, dst):
    return functools.partial(
        pl.kernel,
        out_type=[jax.ShapeDtypeStruct((E,), jnp.float32),
                  jax.ShapeDtypeStruct((NC * NS * NP,), jnp.float32)],
        mesh=_sc_mesh(),
        scratch_types=[
            pltpu.VMEM((CB,), jnp.int32),
            pltpu.VMEM((CB,), jnp.float32),
            pltpu.VMEM((CB,), jnp.float32),
            pltpu.VMEM((CB,), jnp.float32),
            pltpu.VMEM((NP,), jnp.float32),
        ],
        compiler_params=_SC_PARAMS,
    )(_sc_logits_max_body)(part, dst)


def _sc_ex_den_body(logits, dst, mpart, ex, denpart,
                    idst_v, l_v, ex_v, m_v, t_v, den_sp, z_v):
    c = lax.axis_index("c")
    s = lax.axis_index("s")
    wid = s * NC + c
    stripe = NP // NS

    def initz(u, _):
        z_v[pl.ds(u * L, L)] = jnp.zeros((L,), jnp.float32)
        return _
    lax.fori_loop(0, stripe // L, initz, None)
    pltpu.sync_copy(z_v, den_sp.at[pl.ds(pl.multiple_of(s * stripe, 8), stripe)])

    pltpu.sync_copy(mpart.at[pl.ds(0, NP)], m_v)
    def mergerow(w, _):
        pltpu.sync_copy(mpart.at[pl.ds(pl.multiple_of(w * NP, 8), NP)], t_v)

        def vmax(u, _):
            ds_ = pl.ds(u * L, L)
            m_v[ds_] = jnp.maximum(m_v[ds_], t_v[ds_])
            return _
        lax.fori_loop(0, NP // L, vmax, None)
        return _
    lax.fori_loop(1, NC * NS, mergerow, None)
    plsc.subcore_barrier()

    def chunk(i, _):
        t = i * (NC * NS) + wid
        off = pl.multiple_of(t * CD, 8)
        pltpu.sync_copy(logits.at[pl.ds(off, CD)], l_v)
        pltpu.sync_copy(dst.at[pl.ds(off, CD)], idst_v)

        def vex(u, _):
            ds_ = pl.ds(u * L, L)
            iv = idst_v[ds_]
            mg = plsc.load_gather(m_v, [iv])
            ex_v[ds_] = jnp.exp(l_v[ds_] - mg)
            return _
        lax.fori_loop(0, CD // L, vex, None)
        pltpu.sync_copy(ex_v, ex.at[pl.ds(off, CD)])
        pltpu.sync_copy(ex_v, den_sp.at[idst_v], add=True)
        return _

    nchunks = (E // CD - wid + NC * NS - 1) // (NC * NS)
    lax.fori_loop(0, nchunks, chunk, None)
    plsc.subcore_barrier()

    @pl.when(s == 0)
    def _():
        pltpu.sync_copy(den_sp, denpart.at[pl.ds(pl.multiple_of(c * NP, 8), NP)])


def _sc_ex_den(logits, dst, mpart):
    return functools.partial(
        pl.kernel,
        out_type=[jax.ShapeDtypeStruct((E,), jnp.float32),
                  jax.ShapeDtypeStruct((NC * NP,), jnp.float32)],
        mesh=_sc_mesh(),
        scratch_types=[
            pltpu.VMEM((CD,), jnp.int32),
            pltpu.VMEM((CD,), jnp.float32),
            pltpu.VMEM((CD,), jnp.float32),
            pltpu.VMEM((NP,), jnp.float32),
            pltpu.VMEM((NP,), jnp.float32),
            pltpu.VMEM_SHARED((NP,), jnp.float32),
            pltpu.VMEM((NP // NS,), jnp.float32),
        ],
        compiler_params=_SC_PARAMS,
    )(_sc_ex_den_body)(logits, dst, mpart)


def _sc_out_body(vs, es, exv, denpart, src, dst, outf,
                 is0, id0, v0, e0, x0, is1, id1, v1, e1, x1,
                 den_v, db_v, acc_sp, sv0, sx0, sv1, sx1):
    c = lax.axis_index("c")
    s = lax.axis_index("s")
    coff_n = c * NP
    coff_e = c * E
    nstripe = NP // NS

    def zrow(r, _):
        for j in range(H // L):
            v0[r, pl.ds(j * L, L)] = jnp.zeros((L,), jnp.float32)
        return _
    lax.fori_loop(0, CC, zrow, None)
    for rep in range(nstripe // CC):
        pltpu.sync_copy(v0, acc_sp.at[pl.ds(
            pl.multiple_of(s * nstripe + rep * CC, 8), CC)])

    def dmerge(u, _):
        doff = pl.multiple_of(u * nstripe, 8)
        pltpu.sync_copy(denpart.at[pl.ds(doff, nstripe)], db_v)

        def v0(w, _):
            ds_ = pl.ds(w * L, L)
            den_v[pl.ds(pl.multiple_of(u * nstripe, 8) + w * L, L)] = db_v[ds_]
            return _
        lax.fori_loop(0, nstripe // L, v0, None)
        pltpu.sync_copy(denpart.at[pl.ds(pl.multiple_of(NP + doff, 8), nstripe)],
                        db_v)

        def v1(w, _):
            ds_ = pl.ds(pl.multiple_of(u * nstripe, 8) + w * L, L)
            den_v[ds_] = den_v[ds_] + db_v[pl.ds(w * L, L)]
            return _
        lax.fori_loop(0, nstripe // L, v1, None)
        return _
    lax.fori_loop(0, NP // nstripe, dmerge, None)
    plsc.subcore_barrier()

    sets = ((is0, id0, v0, e0, x0, sv0, sx0),
            (is1, id1, v1, e1, x1, sv1, sx1))

    def issue(t, bs):
        isv, idv, vb, eb, xb, sv, sx = bs
        off = pl.multiple_of(t * CC, 8)
        pltpu.sync_copy(dst.at[pl.ds(off, CC)], idv)
        pltpu.sync_copy(src.at[pl.ds(off, CC)], isv)

        def addoff(u, _):
            ds_ = pl.ds(u * L, L)
            isv[ds_] = isv[ds_] + c * N
            return _
        lax.fori_loop(0, CC // L, addoff, None)
        h = (pltpu.async_copy(vs.at[isv], vb, sv),
             pltpu.async_copy(
                 es.at[pl.ds(pl.multiple_of(coff_e + off, 8), CC)], eb, sx))
        pltpu.sync_copy(exv.at[pl.ds(off, CC)], xb)
        return h

    def compute(t, bs, handles):
        isv, idv, vb, eb, xb, sv, sx = bs
        for h in handles:
            h.wait()

        def valpha(u, _):
            ds_ = pl.ds(u * L, L)
            iv = idv[ds_]
            dg = plsc.load_gather(den_v, [iv])
            xb[ds_] = xb[ds_] / (dg + 1e-16)
            return _
        lax.fori_loop(0, CC // L, valpha, None)

        def row(r, _):
            a = plsc.load_gather(xb, [jnp.full((L,), r, jnp.int32)])
            for j in range(H // L):
                ds_ = pl.ds(j * L, L)
                vb[r, ds_] = (vb[r, ds_] + eb[r, ds_]) * a
            return _
        lax.fori_loop(0, CC, row, None)
        pltpu.sync_copy(vb, acc_sp.at[idv], add=True)

    nch = E // CC // NS

    def body(i, _):
        t0 = (2 * i) * NS + s
        t1 = (2 * i + 1) * NS + s
        h0 = issue(t0, sets[0])
        h1 = issue(t1, sets[1])
        compute(t0, sets[0], h0)
        compute(t1, sets[1], h1)
        return _

    lax.fori_loop(0, nch // 2, body, None)
    plsc.subcore_barrier()
    pltpu.sync_copy(acc_sp.at[pl.ds(pl.multiple_of(s * nstripe, 8), nstripe)],
                    outf.at[pl.ds(pl.multiple_of(coff_n + s * nstripe, 8),
                                  nstripe)])


def _sc_out(vs, es, exv, denpart, src, dst):
    return functools.partial(
        pl.kernel,
        out_type=jax.ShapeDtypeStruct((NC * NP, H), jnp.float32),
        mesh=_sc_mesh(),
        scratch_types=[
            pltpu.VMEM((CC,), jnp.int32),
            pltpu.VMEM((CC,), jnp.int32),
            pltpu.VMEM((CC, H), jnp.float32),
            pltpu.VMEM((CC, H), jnp.float32),
            pltpu.VMEM((CC,), jnp.float32),
            pltpu.VMEM((CC,), jnp.int32),
            pltpu.VMEM((CC,), jnp.int32),
            pltpu.VMEM((CC, H), jnp.float32),
            pltpu.VMEM((CC, H), jnp.float32),
            pltpu.VMEM((CC,), jnp.float32),
            pltpu.VMEM((NP,), jnp.float32),
            pltpu.VMEM((NP // NS,), jnp.float32),
            pltpu.VMEM_SHARED((NP, H), jnp.float32),
            pltpu.SemaphoreType.DMA,
            pltpu.SemaphoreType.DMA,
            pltpu.SemaphoreType.DMA,
            pltpu.SemaphoreType.DMA,
        ],
        compiler_params=_SC_PARAMS,
    )(_sc_out_body)(vs, es, exv, denpart, src, dst)


def _tconv(p, h, ef, src, dst):
    q, k, v, s4 = _qkvs(h, p)
    es = _esplit(ef, p['We'], p['be'])
    part = _sc_dots(q, k, es, src, dst)
    logits, mpart = _sc_logits_max(part, dst)
    ex, denpart = _sc_ex_den(logits, dst, mpart)
    outf = _sc_out(v, es, ex, denpart, src, dst)
    return _merge(outf[:N], outf[NP:NP + N], s4)


def kernel(x, edge_index, edge_attr, batch, num_osda, params):
    af, ppe = _embed(x, params)
    ef = _edge_feat(edge_attr, params)
    src = edge_index[0]
    dst = edge_index[1]
    h = _tconv(params['tc'][0], af, ef, src, dst)
    h, ppe = _ppeconv(params['ppe'][0], h, ppe)
    h = _tconv(params['tc'][1], h, ef, src, dst)
    h, ppe = _ppeconv(params['ppe'][1], h, ppe)
    h = _tconv(params['tc'][2], h, ef, src, dst)
    h, ppe = _ppeconv(params['ppe'][2], h, ppe)
    h = _tconv(params['tc'][3], h, ef, src, dst)
    return _pool(h, batch, num_osda, params)
